# Initial kernel scaffold; baseline (speedup 1.0000x reference)
import jax
import jax.numpy as jnp
from jax.experimental import pallas as pl


def kernel(x_nh, output_coords, W_amp, W_wl, dist_weights_phi, dists_0, phi_0):
    raise NotImplementedError("write your pallas kernel here")


# speedup vs baseline: 1.7384x; 1.7384x over previous
import functools

import jax
import jax.numpy as jnp
import numpy as np
from jax.experimental import pallas as pl
from jax.experimental.pallas import tpu as pltpu

PI = float(np.pi)
MIN_DIST = 0.01
MIN_VAL = MIN_DIST / 10.0
HIGHEST = jax.lax.Precision.HIGHEST


def _mm(a, b):
    return jax.lax.dot_general(a, b, (((a.ndim - 1,), (0,)), ((), ())),
                               precision=HIGHEST)


def _acos(z):
    z = jnp.clip(z, -1.0, 1.0)
    return jnp.arctan2(jnp.sqrt((1.0 - z) * (1.0 + z)), z)


def _proj_kernel(w_ref, coeff_ref, ampw_ref, s_ref, st_ref, p_ref, k_ref,
                 cosphi_ref, x_ref, c_ref, o_ref):
    f = 32
    nd = 6
    S = s_ref[...]
    St = st_ref[...]
    P = p_ref[...]
    K = k_ref[...]

    xs = [x_ref[:, d * 128:(d + 1) * 128] for d in range(nd)]

    x_offset = _mm(xs[0], S) * 0.25

    dpre = w_ref[0, 0] * xs[1]
    for d in range(1, nd - 1):
        dpre = dpre + w_ref[0, d] * xs[d + 1]
    e = jnp.exp(dpre)
    dw = e * _mm(1.0 / _mm(e, S), St)

    dirs = _mm(dw * cosphi_ref[...], S)
    direction = _acos(dirs) + PI / 2

    r = _mm(dw, P)
    xw = [_mm(xs[d] * r, S) for d in range(nd)]

    ew = [jnp.exp(t) for t in xw]
    es = ew[0]
    for d in range(1, nd):
        es = es + ew[d]
    wl_pre = coeff_ref[0, 0] * ew[0]
    amp = ampw_ref[0, 0] * (xw[0] - x_offset)
    for d in range(1, nd):
        wl_pre = wl_pre + coeff_ref[0, d] * ew[d]
        amp = amp + ampw_ref[0, d] * (xw[d] - x_offset)
    wl = jax.nn.sigmoid(wl_pre / es) + MIN_VAL

    cr = c_ref[...]
    lat1 = cr[0, :, 0:1]
    lon1 = cr[1, :, 0:1]
    lat2 = cr[0]
    lon2 = cr[1]
    dlon = lon2 - lon1
    sin_lat1 = jnp.sin(lat1)
    cos_lat1 = jnp.cos(lat1)
    sin_lat2 = jnp.sin(lat2)
    cos_lat2 = jnp.cos(lat2)
    cos_dlon = jnp.cos(dlon)
    cosd = sin_lat1 * sin_lat2 + cos_lat1 * cos_lat2 * cos_dlon
    dist = _acos(cosd)
    phis = jnp.arctan2(jnp.sin(dlon) * cos_lat2,
                       cos_lat1 * sin_lat2 - sin_lat1 * cos_lat2 * cos_dlon)

    dist128 = _mm(dist, K)
    phi128 = _mm(phis, K)
    arg = _mm((2.0 * PI) / wl, St) \
        * jnp.cos(_mm(direction, St) - phi128) * dist128
    o_ref[...] = _mm(amp, St) * jnp.cos(arg) + _mm(x_offset, St)


@jax.jit
def kernel(x_nh, output_coords, W_amp, W_wl, dist_weights_phi, dists_0, phi_0):
    b, n, nd, nvm, f = x_nh.shape
    bn = b * n
    k4 = output_coords.shape[-1] // bn
    lanes = nvm * f

    x = x_nh.reshape(bn, nd * lanes)
    coords = output_coords.reshape(2, bn, k4)

    w_soft = jax.nn.softmax(dist_weights_phi).reshape(1, nd - 1)
    coeff = (dists_0[:, 0] * W_wl[0]).reshape(1, nd)
    ampw = W_amp.reshape(1, nd)
    cosphi128 = jnp.repeat(jnp.cos(phi_0[0]), f).reshape(1, lanes)

    j = np.arange(lanes)
    S = np.zeros((lanes, f), np.float32)
    S[j, j % f] = 1.0
    St = S.T.copy()
    P = np.zeros((lanes, lanes), np.float32)
    P[(j % nvm) * f + j // nvm, j] = 1.0
    K = np.zeros((k4, lanes), np.float32)
    K[j // f, j] = 1.0
    S, St, P, K = jnp.asarray(S), jnp.asarray(St), jnp.asarray(P), jnp.asarray(K)

    blk = 8
    for cand in range(min(1024, bn), 0, -1):
        if bn % cand == 0 and cand % 8 == 0:
            blk = cand
            break
    grid = (bn // blk,)

    smem = pl.BlockSpec(memory_space=pltpu.SMEM)
    out = pl.pallas_call(
        _proj_kernel,
        grid=grid,
        in_specs=[
            smem, smem, smem,
            pl.BlockSpec((lanes, f), lambda i: (0, 0)),
            pl.BlockSpec((f, lanes), lambda i: (0, 0)),
            pl.BlockSpec((lanes, lanes), lambda i: (0, 0)),
            pl.BlockSpec((k4, lanes), lambda i: (0, 0)),
            pl.BlockSpec((1, lanes), lambda i: (0, 0)),
            pl.BlockSpec((blk, nd * lanes), lambda i: (i, 0)),
            pl.BlockSpec((2, blk, k4), lambda i: (0, i, 0)),
        ],
        out_specs=pl.BlockSpec((blk, k4 * f), lambda i: (i, 0)),
        out_shape=jax.ShapeDtypeStruct((bn, k4 * f), x.dtype),
    )(w_soft, coeff, ampw, S, St, P, K, cosphi128, x, coords)
    return out.reshape(b, n * k4, f)
